# baseline (speedup 1.0000x reference)
import functools

import jax
import jax.numpy as jnp
from jax import lax
from jax.experimental import pallas as pl
from jax.experimental.pallas import tpu as pltpu
from jax.experimental.pallas import tpu_sc as plsc

NUM_COMPONENTS = 64
BATCH = 16384
EPS = 1e-6

NC = 2
NS = 16
NW = NC * NS
B_PER_W = BATCH // NW
NCHUNK = 4
CHUNK = B_PER_W // NCHUNK
NGROUP = CHUNK // 16
RING = 2
PAIR_W = 2 * NUM_COMPONENTS


def _sqrt16(x):
    i = lax.bitcast_convert_type(x, jnp.int32)
    i = 0x5F3759DF - lax.shift_right_logical(i, 1)
    y = lax.bitcast_convert_type(i, jnp.float32)
    for _ in range(3):
        y = y * (1.5 - 0.5 * x * y * y)
    return x * y


def _sc_body(Uh_hbm, Ih_hbm, Uo_hbm, Io_hbm, W2_hbm, H2_hbm, out_hbm,
             idx_u, idx_i, osel_u, osel_i,
             w0, w1, h0, h1, out_v, semw, semh):
    wslot = (w0, w1)
    hslot = (h0, h1)
    wid = lax.axis_index("s") * NC + lax.axis_index("c")
    pltpu.sync_copy(Uh_hbm.at[wid], idx_u)
    pltpu.sync_copy(Ih_hbm.at[wid], idx_i)
    pltpu.sync_copy(Uo_hbm.at[wid], osel_u)
    pltpu.sync_copy(Io_hbm.at[wid], osel_i)

    def fire(k):
        s = k % RING
        cw = pltpu.async_copy(W2_hbm.at[idx_u.at[pl.ds(k * CHUNK, CHUNK)]],
                              wslot[s], semw.at[s])
        ch = pltpu.async_copy(H2_hbm.at[idx_i.at[pl.ds(k * CHUNK, CHUNK)]],
                              hslot[s], semh.at[s])
        return cw, ch

    cps = [None] * NCHUNK
    for k in range(RING):
        cps[k] = fire(k)
    for k in range(NCHUNK):
        s = k % RING
        cps[k][0].wait()
        cps[k][1].wait()
        wbuf, hbuf = wslot[s], hslot[s]

        def group(g, _, k=k, wbuf=wbuf, hbuf=hbuf):
            rows = g * 16 + lax.broadcasted_iota(jnp.int32, (16,), 0)
            off = k * CHUNK + g * 16
            ou = osel_u[pl.ds(off, 16)]
            oi = osel_i[pl.ds(off, 16)]
            acc = jnp.zeros((16,), jnp.float32)
            for c in range(NUM_COMPONENTS):
                wv = plsc.load_gather(wbuf, [rows, ou + c])
                hv = plsc.load_gather(hbuf, [rows, oi + c])
                d = wv - hv + EPS
                acc = acc + d * d
            out_v[pl.ds(off, 16)] = _sqrt16(acc)
            return 0

        lax.fori_loop(0, NGROUP, group, 0)
        if k + RING < NCHUNK:
            cps[k + RING] = fire(k + RING)
    pltpu.sync_copy(out_v, out_hbm.at[wid])


@jax.jit
def kernel(U, I, W, H):
    U32 = U.astype(jnp.int32)
    I32 = I.astype(jnp.int32)
    Uh = lax.shift_right_logical(U32, 1).reshape(NW, B_PER_W)
    Ih = lax.shift_right_logical(I32, 1).reshape(NW, B_PER_W)
    Uo = ((U32 & 1) * NUM_COMPONENTS).reshape(NW, B_PER_W)
    Io = ((I32 & 1) * NUM_COMPONENTS).reshape(NW, B_PER_W)
    W2 = W.astype(jnp.bfloat16).astype(jnp.float32).reshape(W.shape[0] // 2, PAIR_W)
    H2 = H.astype(jnp.bfloat16).astype(jnp.float32).reshape(H.shape[0] // 2, PAIR_W)
    mesh = plsc.VectorSubcoreMesh(core_axis_name="c", subcore_axis_name="s")
    run = functools.partial(
        pl.kernel,
        mesh=mesh,
        compiler_params=pltpu.CompilerParams(needs_layout_passes=False),
        out_type=jax.ShapeDtypeStruct((NW, B_PER_W), jnp.float32),
        scratch_types=[
            pltpu.VMEM((B_PER_W,), jnp.int32),
            pltpu.VMEM((B_PER_W,), jnp.int32),
            pltpu.VMEM((B_PER_W,), jnp.int32),
            pltpu.VMEM((B_PER_W,), jnp.int32),
            pltpu.VMEM((CHUNK, PAIR_W), jnp.float32),
            pltpu.VMEM((CHUNK, PAIR_W), jnp.float32),
            pltpu.VMEM((CHUNK, PAIR_W), jnp.float32),
            pltpu.VMEM((CHUNK, PAIR_W), jnp.float32),
            pltpu.VMEM((B_PER_W,), jnp.float32),
            pltpu.SemaphoreType.DMA((RING,)),
            pltpu.SemaphoreType.DMA((RING,)),
        ],
    )(_sc_body)
    out = run(Uh, Ih, Uo, Io, W2, H2)
    return out.reshape(BATCH)


# speedup vs baseline: 2.0182x; 2.0182x over previous
import functools

import jax
import jax.numpy as jnp
from jax import lax
from jax.experimental import pallas as pl
from jax.experimental.pallas import tpu as pltpu
from jax.experimental.pallas import tpu_sc as plsc

NUM_COMPONENTS = 64
NUSERS = 1000000
BATCH = 16384
EPS = 1e-6

NC = 2
NS = 16
NW = NC * NS
B_PER_W = BATCH // NW
NCHUNK = 4
CHUNK = B_PER_W // NCHUNK
NGROUP = CHUNK // 16
RING = 2
GW = 2 * NUM_COMPONENTS

TBLK = 2048
TGRID = (NUSERS + TBLK - 1) // TBLK


def _tp_body(wt_ref, ht_ref, out_ref):
    out_ref[:, :NUM_COMPONENTS] = wt_ref[...].T
    out_ref[:, NUM_COMPONENTS:] = ht_ref[...].T


def _build_combined(WT, HT):
    return pl.pallas_call(
        _tp_body,
        grid=(TGRID,),
        in_specs=[
            pl.BlockSpec((NUM_COMPONENTS, TBLK), lambda i: (0, i)),
            pl.BlockSpec((NUM_COMPONENTS, TBLK), lambda i: (0, i)),
        ],
        out_specs=pl.BlockSpec((TBLK, GW), lambda i: (i, 0)),
        out_shape=jax.ShapeDtypeStruct((NUSERS, GW), jnp.float32),
    )(WT, HT)


def _sqrt16(x):
    i = lax.bitcast_convert_type(x, jnp.int32)
    i = 0x5F3759DF - lax.shift_right_logical(i, 1)
    y = lax.bitcast_convert_type(i, jnp.float32)
    for _ in range(3):
        y = y * (1.5 - 0.5 * x * y * y)
    return x * y


def _sc_body(U_hbm, I_hbm, G_hbm, out_hbm,
             idx_u, idx_i, w0, w1, h0, h1, out_v, semw, semh):
    wslot = (w0, w1)
    hslot = (h0, h1)
    wid = lax.axis_index("s") * NC + lax.axis_index("c")
    pltpu.sync_copy(U_hbm.at[wid], idx_u)
    pltpu.sync_copy(I_hbm.at[wid], idx_i)

    def fire(k):
        s = k % RING
        cw = pltpu.async_copy(G_hbm.at[idx_u.at[pl.ds(k * CHUNK, CHUNK)]],
                              wslot[s], semw.at[s])
        ch = pltpu.async_copy(G_hbm.at[idx_i.at[pl.ds(k * CHUNK, CHUNK)]],
                              hslot[s], semh.at[s])
        return cw, ch

    cps = [None] * NCHUNK
    for k in range(RING):
        cps[k] = fire(k)
    for k in range(NCHUNK):
        s = k % RING
        cps[k][0].wait()
        cps[k][1].wait()
        wbuf, hbuf = wslot[s], hslot[s]

        def group(g, _, k=k, wbuf=wbuf, hbuf=hbuf):
            rows = g * 16 + lax.broadcasted_iota(jnp.int32, (16,), 0)
            off = k * CHUNK + g * 16
            acc = jnp.zeros((16,), jnp.float32)
            for c in range(NUM_COMPONENTS):
                wv = plsc.load_gather(wbuf, [rows, jnp.full((16,), c,
                                                            jnp.int32)])
                hv = plsc.load_gather(hbuf, [rows,
                                             jnp.full((16,),
                                                      NUM_COMPONENTS + c,
                                                      jnp.int32)])
                d = wv - hv + EPS
                acc = acc + d * d
            out_v[pl.ds(off, 16)] = _sqrt16(acc)
            return 0

        lax.fori_loop(0, NGROUP, group, 0)
        if k + RING < NCHUNK:
            cps[k + RING] = fire(k + RING)
    pltpu.sync_copy(out_v, out_hbm.at[wid])


@jax.jit
def kernel(U, I, W, H):
    U2 = U.astype(jnp.int32).reshape(NW, B_PER_W)
    I2 = I.astype(jnp.int32).reshape(NW, B_PER_W)
    G = _build_combined(W.T, H.T)
    mesh = plsc.VectorSubcoreMesh(core_axis_name="c", subcore_axis_name="s")
    run = functools.partial(
        pl.kernel,
        mesh=mesh,
        compiler_params=pltpu.CompilerParams(needs_layout_passes=False),
        out_type=jax.ShapeDtypeStruct((NW, B_PER_W), jnp.float32),
        scratch_types=[
            pltpu.VMEM((B_PER_W,), jnp.int32),
            pltpu.VMEM((B_PER_W,), jnp.int32),
            pltpu.VMEM((CHUNK, GW), jnp.float32),
            pltpu.VMEM((CHUNK, GW), jnp.float32),
            pltpu.VMEM((CHUNK, GW), jnp.float32),
            pltpu.VMEM((CHUNK, GW), jnp.float32),
            pltpu.VMEM((B_PER_W,), jnp.float32),
            pltpu.SemaphoreType.DMA((RING,)),
            pltpu.SemaphoreType.DMA((RING,)),
        ],
    )(_sc_body)
    out = run(U2, I2, G)
    return out.reshape(BATCH)


# speedup vs baseline: 3.7570x; 1.8616x over previous
import functools

import jax
import jax.numpy as jnp
from jax import lax
from jax.experimental import pallas as pl
from jax.experimental.pallas import tpu as pltpu
from jax.experimental.pallas import tpu_sc as plsc

NUM_COMPONENTS = 64
NUSERS = 1000000
BATCH = 16384
EPS = 1e-6

NC = 2
NS = 16
NW = NC * NS
B_PER_W = BATCH // NW
NCHUNK = 4
CHUNK = B_PER_W // NCHUNK
NGROUP = CHUNK // 16
RING = 2
GW = 2 * NUM_COMPONENTS

TBLK = 16384
TGRID = (NUSERS + TBLK - 1) // TBLK


def _tp_body(wt_ref, ht_ref, out_ref):
    r = lax.broadcasted_iota(jnp.int32, (GW, GW), 0)
    c = lax.broadcasted_iota(jnp.int32, (GW, GW), 1)
    ident = jnp.where(r == c, 1.0, 0.0).astype(jnp.float32)
    lhs = jnp.concatenate([wt_ref[...], ht_ref[...]], axis=0)
    out_ref[...] = lax.dot_general(lhs, ident,
                                   dimension_numbers=(((0,), (0,)), ((), ())),
                                   preferred_element_type=jnp.float32)


def _build_combined(WT, HT):
    return pl.pallas_call(
        _tp_body,
        grid=(TGRID,),
        in_specs=[
            pl.BlockSpec((NUM_COMPONENTS, TBLK), lambda i: (0, i)),
            pl.BlockSpec((NUM_COMPONENTS, TBLK), lambda i: (0, i)),
        ],
        out_specs=pl.BlockSpec((TBLK, GW), lambda i: (i, 0)),
        out_shape=jax.ShapeDtypeStruct((NUSERS, GW), jnp.float32),
    )(WT, HT)


def _sqrt16(x):
    i = lax.bitcast_convert_type(x, jnp.int32)
    i = 0x5F3759DF - lax.shift_right_logical(i, 1)
    y = lax.bitcast_convert_type(i, jnp.float32)
    for _ in range(3):
        y = y * (1.5 - 0.5 * x * y * y)
    return x * y


def _sc_body(U_hbm, I_hbm, G_hbm, out_hbm,
             idx_u, idx_i, w0, w1, h0, h1, out_v, semw, semh):
    wslot = (w0, w1)
    hslot = (h0, h1)
    wid = lax.axis_index("s") * NC + lax.axis_index("c")
    pltpu.sync_copy(U_hbm.at[wid], idx_u)
    pltpu.sync_copy(I_hbm.at[wid], idx_i)

    def fire(k):
        s = k % RING
        cw = pltpu.async_copy(G_hbm.at[idx_u.at[pl.ds(k * CHUNK, CHUNK)]],
                              wslot[s], semw.at[s])
        ch = pltpu.async_copy(G_hbm.at[idx_i.at[pl.ds(k * CHUNK, CHUNK)]],
                              hslot[s], semh.at[s])
        return cw, ch

    cps = [None] * NCHUNK
    for k in range(RING):
        cps[k] = fire(k)
    for k in range(NCHUNK):
        s = k % RING
        cps[k][0].wait()
        cps[k][1].wait()
        wbuf, hbuf = wslot[s], hslot[s]

        def group(g, _, k=k, wbuf=wbuf, hbuf=hbuf):
            rows = g * 16 + lax.broadcasted_iota(jnp.int32, (16,), 0)
            off = k * CHUNK + g * 16
            acc = jnp.zeros((16,), jnp.float32)
            for c in range(NUM_COMPONENTS):
                wv = plsc.load_gather(wbuf, [rows, jnp.full((16,), c,
                                                            jnp.int32)])
                hv = plsc.load_gather(hbuf, [rows,
                                             jnp.full((16,),
                                                      NUM_COMPONENTS + c,
                                                      jnp.int32)])
                d = wv - hv + EPS
                acc = acc + d * d
            out_v[pl.ds(off, 16)] = _sqrt16(acc)
            return 0

        lax.fori_loop(0, NGROUP, group, 0)
        if k + RING < NCHUNK:
            cps[k + RING] = fire(k + RING)
    pltpu.sync_copy(out_v, out_hbm.at[wid])


@jax.jit
def kernel(U, I, W, H):
    U2 = U.astype(jnp.int32).reshape(NW, B_PER_W)
    I2 = I.astype(jnp.int32).reshape(NW, B_PER_W)
    G = _build_combined(W.T, H.T)
    mesh = plsc.VectorSubcoreMesh(core_axis_name="c", subcore_axis_name="s")
    run = functools.partial(
        pl.kernel,
        mesh=mesh,
        compiler_params=pltpu.CompilerParams(needs_layout_passes=False),
        out_type=jax.ShapeDtypeStruct((NW, B_PER_W), jnp.float32),
        scratch_types=[
            pltpu.VMEM((B_PER_W,), jnp.int32),
            pltpu.VMEM((B_PER_W,), jnp.int32),
            pltpu.VMEM((CHUNK, GW), jnp.float32),
            pltpu.VMEM((CHUNK, GW), jnp.float32),
            pltpu.VMEM((CHUNK, GW), jnp.float32),
            pltpu.VMEM((CHUNK, GW), jnp.float32),
            pltpu.VMEM((B_PER_W,), jnp.float32),
            pltpu.SemaphoreType.DMA((RING,)),
            pltpu.SemaphoreType.DMA((RING,)),
        ],
    )(_sc_body)
    out = run(U2, I2, G)
    return out.reshape(BATCH)


# speedup vs baseline: 3.7737x; 1.0044x over previous
import functools

import jax
import jax.numpy as jnp
from jax import lax
from jax.experimental import pallas as pl
from jax.experimental.pallas import tpu as pltpu
from jax.experimental.pallas import tpu_sc as plsc

NUM_COMPONENTS = 64
NUSERS = 1000000
BATCH = 16384
EPS = 1e-6

NC = 2
NS = 16
NW = NC * NS
B_PER_W = BATCH // NW
NCHUNK = 4
CHUNK = B_PER_W // NCHUNK
NGROUP = CHUNK // 16
RING = 2
GW = 2 * NUM_COMPONENTS

TBLK = 24576
TGRID = (NUSERS + TBLK - 1) // TBLK


def _tp_body(wt_ref, ht_ref, out_ref):
    r = lax.broadcasted_iota(jnp.int32, (GW, GW), 0)
    c = lax.broadcasted_iota(jnp.int32, (GW, GW), 1)
    ident = jnp.where(r == c, 1.0, 0.0).astype(jnp.float32)
    lhs = jnp.concatenate([wt_ref[...], ht_ref[...]], axis=0)
    out_ref[...] = lax.dot_general(lhs, ident,
                                   dimension_numbers=(((0,), (0,)), ((), ())),
                                   preferred_element_type=jnp.float32)


def _build_combined(WT, HT):
    return pl.pallas_call(
        _tp_body,
        grid=(TGRID,),
        in_specs=[
            pl.BlockSpec((NUM_COMPONENTS, TBLK), lambda i: (0, i)),
            pl.BlockSpec((NUM_COMPONENTS, TBLK), lambda i: (0, i)),
        ],
        out_specs=pl.BlockSpec((TBLK, GW), lambda i: (i, 0)),
        out_shape=jax.ShapeDtypeStruct((NUSERS, GW), jnp.float32),
    )(WT, HT)


def _sqrt16(x):
    i = lax.bitcast_convert_type(x, jnp.int32)
    i = 0x5F3759DF - lax.shift_right_logical(i, 1)
    y = lax.bitcast_convert_type(i, jnp.float32)
    for _ in range(3):
        y = y * (1.5 - 0.5 * x * y * y)
    return x * y


def _sc_body(U_hbm, I_hbm, G_hbm, out_hbm,
             idx_u, idx_i, w0, w1, h0, h1, out_v, semw, semh):
    wslot = (w0, w1)
    hslot = (h0, h1)
    wid = lax.axis_index("s") * NC + lax.axis_index("c")
    pltpu.sync_copy(U_hbm.at[wid], idx_u)
    pltpu.sync_copy(I_hbm.at[wid], idx_i)

    def fire(k):
        s = k % RING
        cw = pltpu.async_copy(G_hbm.at[idx_u.at[pl.ds(k * CHUNK, CHUNK)]],
                              wslot[s], semw.at[s])
        ch = pltpu.async_copy(G_hbm.at[idx_i.at[pl.ds(k * CHUNK, CHUNK)]],
                              hslot[s], semh.at[s])
        return cw, ch

    cps = [None] * NCHUNK
    for k in range(RING):
        cps[k] = fire(k)
    for k in range(NCHUNK):
        s = k % RING
        cps[k][0].wait()
        cps[k][1].wait()
        wbuf, hbuf = wslot[s], hslot[s]

        def group(g, _, k=k, wbuf=wbuf, hbuf=hbuf):
            rows = g * 16 + lax.broadcasted_iota(jnp.int32, (16,), 0)
            off = k * CHUNK + g * 16
            acc = jnp.zeros((16,), jnp.float32)
            for c in range(NUM_COMPONENTS):
                wv = plsc.load_gather(wbuf, [rows, jnp.full((16,), c,
                                                            jnp.int32)])
                hv = plsc.load_gather(hbuf, [rows,
                                             jnp.full((16,),
                                                      NUM_COMPONENTS + c,
                                                      jnp.int32)])
                d = wv - hv + EPS
                acc = acc + d * d
            out_v[pl.ds(off, 16)] = _sqrt16(acc)
            return 0

        lax.fori_loop(0, NGROUP, group, 0)
        if k + RING < NCHUNK:
            cps[k + RING] = fire(k + RING)
    pltpu.sync_copy(out_v, out_hbm.at[wid])


@jax.jit
def kernel(U, I, W, H):
    U2 = U.astype(jnp.int32).reshape(NW, B_PER_W)
    I2 = I.astype(jnp.int32).reshape(NW, B_PER_W)
    G = _build_combined(W.T, H.T)
    mesh = plsc.VectorSubcoreMesh(core_axis_name="c", subcore_axis_name="s")
    run = functools.partial(
        pl.kernel,
        mesh=mesh,
        compiler_params=pltpu.CompilerParams(needs_layout_passes=False),
        out_type=jax.ShapeDtypeStruct((NW, B_PER_W), jnp.float32),
        scratch_types=[
            pltpu.VMEM((B_PER_W,), jnp.int32),
            pltpu.VMEM((B_PER_W,), jnp.int32),
            pltpu.VMEM((CHUNK, GW), jnp.float32),
            pltpu.VMEM((CHUNK, GW), jnp.float32),
            pltpu.VMEM((CHUNK, GW), jnp.float32),
            pltpu.VMEM((CHUNK, GW), jnp.float32),
            pltpu.VMEM((B_PER_W,), jnp.float32),
            pltpu.SemaphoreType.DMA((RING,)),
            pltpu.SemaphoreType.DMA((RING,)),
        ],
    )(_sc_body)
    out = run(U2, I2, G)
    return out.reshape(BATCH)


# speedup vs baseline: 3.7750x; 1.0004x over previous
import functools

import jax
import jax.numpy as jnp
from jax import lax
from jax.experimental import pallas as pl
from jax.experimental.pallas import tpu as pltpu
from jax.experimental.pallas import tpu_sc as plsc

NUM_COMPONENTS = 64
NUSERS = 1000000
BATCH = 16384
EPS = 1e-6

NC = 2
NS = 16
NW = NC * NS
B_PER_W = BATCH // NW
NCHUNK = 8
CHUNK = B_PER_W // NCHUNK
NGROUP = CHUNK // 16
RING = 4
GW = 2 * NUM_COMPONENTS

TBLK = 24576
TGRID = (NUSERS + TBLK - 1) // TBLK


def _tp_body(wt_ref, ht_ref, out_ref):
    r = lax.broadcasted_iota(jnp.int32, (GW, GW), 0)
    c = lax.broadcasted_iota(jnp.int32, (GW, GW), 1)
    ident = jnp.where(r == c, 1.0, 0.0).astype(jnp.float32)
    lhs = jnp.concatenate([wt_ref[...], ht_ref[...]], axis=0)
    out_ref[...] = lax.dot_general(lhs, ident,
                                   dimension_numbers=(((0,), (0,)), ((), ())),
                                   preferred_element_type=jnp.float32)


def _build_combined(WT, HT):
    return pl.pallas_call(
        _tp_body,
        grid=(TGRID,),
        in_specs=[
            pl.BlockSpec((NUM_COMPONENTS, TBLK), lambda i: (0, i)),
            pl.BlockSpec((NUM_COMPONENTS, TBLK), lambda i: (0, i)),
        ],
        out_specs=pl.BlockSpec((TBLK, GW), lambda i: (i, 0)),
        out_shape=jax.ShapeDtypeStruct((NUSERS, GW), jnp.float32),
    )(WT, HT)


def _sqrt16(x):
    i = lax.bitcast_convert_type(x, jnp.int32)
    i = 0x5F3759DF - lax.shift_right_logical(i, 1)
    y = lax.bitcast_convert_type(i, jnp.float32)
    for _ in range(3):
        y = y * (1.5 - 0.5 * x * y * y)
    return x * y


def _sc_body(U_hbm, I_hbm, G_hbm, out_hbm,
             idx_u, idx_i, w0, w1, w2, w3, h0, h1, h2, h3, out_v, semw, semh):
    wslot = (w0, w1, w2, w3)
    hslot = (h0, h1, h2, h3)
    wid = lax.axis_index("s") * NC + lax.axis_index("c")
    pltpu.sync_copy(U_hbm.at[wid], idx_u)
    pltpu.sync_copy(I_hbm.at[wid], idx_i)

    def fire(k):
        s = k % RING
        cw = pltpu.async_copy(G_hbm.at[idx_u.at[pl.ds(k * CHUNK, CHUNK)]],
                              wslot[s], semw.at[s])
        ch = pltpu.async_copy(G_hbm.at[idx_i.at[pl.ds(k * CHUNK, CHUNK)]],
                              hslot[s], semh.at[s])
        return cw, ch

    cps = [None] * NCHUNK
    for k in range(RING):
        cps[k] = fire(k)
    for k in range(NCHUNK):
        s = k % RING
        cps[k][0].wait()
        cps[k][1].wait()
        wbuf, hbuf = wslot[s], hslot[s]

        def group(g, _, k=k, wbuf=wbuf, hbuf=hbuf):
            rows = g * 16 + lax.broadcasted_iota(jnp.int32, (16,), 0)
            off = k * CHUNK + g * 16
            acc = jnp.zeros((16,), jnp.float32)
            for c in range(NUM_COMPONENTS):
                wv = plsc.load_gather(wbuf, [rows, jnp.full((16,), c,
                                                            jnp.int32)])
                hv = plsc.load_gather(hbuf, [rows,
                                             jnp.full((16,),
                                                      NUM_COMPONENTS + c,
                                                      jnp.int32)])
                d = wv - hv + EPS
                acc = acc + d * d
            out_v[pl.ds(off, 16)] = _sqrt16(acc)
            return 0

        lax.fori_loop(0, NGROUP, group, 0)
        if k + RING < NCHUNK:
            cps[k + RING] = fire(k + RING)
    pltpu.sync_copy(out_v, out_hbm.at[wid])


@jax.jit
def kernel(U, I, W, H):
    U2 = U.astype(jnp.int32).reshape(NW, B_PER_W)
    I2 = I.astype(jnp.int32).reshape(NW, B_PER_W)
    G = _build_combined(W.T, H.T)
    mesh = plsc.VectorSubcoreMesh(core_axis_name="c", subcore_axis_name="s")
    run = functools.partial(
        pl.kernel,
        mesh=mesh,
        compiler_params=pltpu.CompilerParams(needs_layout_passes=False),
        out_type=jax.ShapeDtypeStruct((NW, B_PER_W), jnp.float32),
        scratch_types=[
            pltpu.VMEM((B_PER_W,), jnp.int32),
            pltpu.VMEM((B_PER_W,), jnp.int32),
            pltpu.VMEM((CHUNK, GW), jnp.float32),
            pltpu.VMEM((CHUNK, GW), jnp.float32),
            pltpu.VMEM((CHUNK, GW), jnp.float32),
            pltpu.VMEM((CHUNK, GW), jnp.float32),
            pltpu.VMEM((CHUNK, GW), jnp.float32),
            pltpu.VMEM((CHUNK, GW), jnp.float32),
            pltpu.VMEM((CHUNK, GW), jnp.float32),
            pltpu.VMEM((CHUNK, GW), jnp.float32),
            pltpu.VMEM((B_PER_W,), jnp.float32),
            pltpu.SemaphoreType.DMA((RING,)),
            pltpu.SemaphoreType.DMA((RING,)),
        ],
    )(_sc_body)
    out = run(U2, I2, G)
    return out.reshape(BATCH)
